# baseline (speedup 1.0000x reference)
import functools

import jax
import jax.numpy as jnp
from jax import lax
from jax.experimental import pallas as pl
from jax.experimental.pallas import tpu as pltpu
from jax.experimental.pallas import tpu_sc as plsc

E = 8
D_MODEL = 1024
D_FF = 704
T = 2048
TM = 256
NW = 32
CHUNK = T // NW
L = 16

_NT = (((1,), (1,)), ((), ()))


def _silu(x):
    return x * jax.nn.sigmoid(x)


def _gate_body(x_ref, gate_ref, logt_ref):
    logt_ref[...] = jax.lax.dot_general(
        gate_ref[...], x_ref[...], _NT, preferred_element_type=jnp.float32)


def _router_body(logt_hbm, coef_hbm, lg_v, cf_v):
    wid = lax.axis_index("s") * 2 + lax.axis_index("c")
    base = wid * CHUNK
    for e in range(E):
        pltpu.sync_copy(logt_hbm.at[pl.ds(e * T + base, CHUNK)],
                        lg_v.at[pl.ds(e * CHUNK, CHUNK)])
    one = jnp.full((L,), 1.0, dtype=jnp.float32)
    zero = jnp.full((L,), 0.0, dtype=jnp.float32)
    neg = jnp.full((L,), -3e38, dtype=jnp.float32)
    for j in range(CHUNK // L):
        v = [lg_v[pl.ds(e * CHUNK + j * L, L)] for e in range(E)]
        m = functools.reduce(jnp.maximum, v)
        ex = [jnp.exp(ve - m) for ve in v]
        s = functools.reduce(lambda a, b: a + b, ex)
        inv = 1.0 / s
        seen = zero
        pick1 = [None] * E
        for e in range(E):
            hit = jnp.minimum(jnp.where(v[e] == m, one, zero), 1.0 - seen)
            pick1[e] = hit
            seen = seen + hit
        v2 = [jnp.where(pick1[e] > 0.5, neg, v[e]) for e in range(E)]
        m2 = functools.reduce(jnp.maximum, v2)
        seen2 = zero
        for e in range(E):
            hit2 = jnp.minimum(jnp.where(v2[e] == m2, one, zero),
                               1.0 - seen2)
            seen2 = seen2 + hit2
            cf_v[pl.ds(e * CHUNK + j * L, L)] = \
                (pick1[e] + hit2) * (ex[e] * inv)
    for e in range(E):
        pltpu.sync_copy(cf_v.at[pl.ds(e * CHUNK, CHUNK)],
                        coef_hbm.at[pl.ds(e * T + base, CHUNK)])


def _moe_body(x_ref, coef_ref, wg_ref, wu_ref, wd_ref, shg_ref, shu_ref,
              shd_ref, out_ref, acc_ref, sem):
    e = pl.program_id(0)
    i = pl.program_id(1)
    rows = pl.ds(i * TM, TM)
    x = x_ref[rows, :]

    @pl.when(e == 0)
    def _shared():
        g = jax.lax.dot_general(x, shg_ref[...], _NT,
                                preferred_element_type=jnp.float32)
        u = jax.lax.dot_general(x, shu_ref[...], _NT,
                                preferred_element_type=jnp.float32)
        t = _silu(g) * u
        acc_ref[rows, :] = jax.lax.dot_general(
            t, shd_ref[...], _NT, preferred_element_type=jnp.float32)

    @pl.when(e > 0)
    def _expert():
        g = jax.lax.dot_general(x, wg_ref[0], _NT,
                                preferred_element_type=jnp.float32)
        u = jax.lax.dot_general(x, wu_ref[0], _NT,
                                preferred_element_type=jnp.float32)
        t = _silu(g) * u
        y = jax.lax.dot_general(t, wd_ref[0], _NT,
                                preferred_element_type=jnp.float32)
        crow = coef_ref[pl.ds(jnp.maximum(e - 1, 0), 1), rows]
        r2 = lax.broadcasted_iota(jnp.int32, (TM, TM), 0)
        c2 = lax.broadcasted_iota(jnp.int32, (TM, TM), 1)
        eye = jnp.where(r2 == c2, 1.0, 0.0).astype(jnp.float32)
        ccol = jax.lax.dot_general(eye, crow, _NT,
                                   precision=jax.lax.Precision.HIGHEST,
                                   preferred_element_type=jnp.float32)
        acc_ref[rows, :] = acc_ref[rows, :] + ccol * y

    @pl.when(e == E)
    def _flush():
        cp = pltpu.make_async_copy(acc_ref.at[rows, :], out_ref.at[rows, :],
                                   sem)
        cp.start()
        cp.wait()


def kernel(hidden_states, gate_w, w_gate, w_up, w_down, sh_gate, sh_up,
           sh_down):
    bsz, seq_len, h = hidden_states.shape
    x = hidden_states.reshape(-1, h)
    d_sh = sh_gate.shape[0]

    logt = pl.pallas_call(
        _gate_body,
        grid=(T // TM,),
        in_specs=[
            pl.BlockSpec((TM, D_MODEL), lambda i: (i, 0)),
            pl.BlockSpec((E, D_MODEL), lambda i: (0, 0)),
        ],
        out_specs=pl.BlockSpec((E, TM), lambda i: (0, i)),
        out_shape=jax.ShapeDtypeStruct((E, T), jnp.float32),
    )(x, gate_w)

    router = functools.partial(
        pl.kernel,
        mesh=plsc.VectorSubcoreMesh(core_axis_name="c", subcore_axis_name="s",
                                    num_cores=2, num_subcores=16),
        out_type=jax.ShapeDtypeStruct((T * E,), jnp.float32),
        scratch_types=[
            pltpu.VMEM((E * CHUNK,), jnp.float32),
            pltpu.VMEM((CHUNK * E,), jnp.float32),
        ],
    )(_router_body)
    coefs_t = router(logt.reshape(E * T)).reshape(E, T)

    ei = lambda e: jnp.maximum(e - 1, 0)
    out = pl.pallas_call(
        _moe_body,
        grid=(E + 1, T // TM),
        in_specs=[
            pl.BlockSpec((T, D_MODEL), lambda e, i: (0, 0)),
            pl.BlockSpec((E, T), lambda e, i: (0, 0)),
            pl.BlockSpec((1, D_FF, D_MODEL), lambda e, i: (ei(e), 0, 0)),
            pl.BlockSpec((1, D_FF, D_MODEL), lambda e, i: (ei(e), 0, 0)),
            pl.BlockSpec((1, D_MODEL, D_FF), lambda e, i: (ei(e), 0, 0)),
            pl.BlockSpec((d_sh, D_MODEL), lambda e, i: (0, 0)),
            pl.BlockSpec((d_sh, D_MODEL), lambda e, i: (0, 0)),
            pl.BlockSpec((D_MODEL, d_sh), lambda e, i: (0, 0)),
        ],
        out_specs=pl.BlockSpec(memory_space=pl.ANY),
        out_shape=jax.ShapeDtypeStruct((T, D_MODEL), jnp.float32),
        scratch_shapes=[
            pltpu.VMEM((T, D_MODEL), jnp.float32),
            pltpu.SemaphoreType.DMA,
        ],
    )(x, coefs_t, w_gate, w_up, w_down, sh_gate, sh_up, sh_down)
    return out.reshape(bsz, seq_len, h).astype(hidden_states.dtype)


# speedup vs baseline: 1.1196x; 1.1196x over previous
import functools

import jax
import jax.numpy as jnp
from jax import lax
from jax.experimental import pallas as pl
from jax.experimental.pallas import tpu as pltpu
from jax.experimental.pallas import tpu_sc as plsc

E = 8
EH = 1
NH = E // EH
D_MODEL = 1024
D_FF = 704
T = 2048
TA = 512
TM = 512
NW = 32
CHUNK = T // NW
L = 16

_NT = (((1,), (1,)), ((), ()))
_NN = (((1,), (0,)), ((), ()))


def _silu(x):
    return x * jax.nn.sigmoid(x)


def _gate_body(x_ref, gate_ref, logt_ref):
    logt_ref[...] = jax.lax.dot_general(
        gate_ref[...], x_ref[...], _NT, preferred_element_type=jnp.float32)


def _router_body(logt_hbm, coef_hbm, lg_v, cf_v):
    wid = lax.axis_index("s") * 2 + lax.axis_index("c")
    base = wid * CHUNK
    for e in range(E):
        pltpu.sync_copy(logt_hbm.at[pl.ds(e * T + base, CHUNK)],
                        lg_v.at[pl.ds(e * CHUNK, CHUNK)])
    one = jnp.full((L,), 1.0, dtype=jnp.float32)
    zero = jnp.full((L,), 0.0, dtype=jnp.float32)
    neg = jnp.full((L,), -3e38, dtype=jnp.float32)
    for j in range(CHUNK // L):
        v = [lg_v[pl.ds(e * CHUNK + j * L, L)] for e in range(E)]
        m = functools.reduce(jnp.maximum, v)
        ex = [jnp.exp(ve - m) for ve in v]
        s = functools.reduce(lambda a, b: a + b, ex)
        inv = 1.0 / s
        seen = zero
        pick1 = [None] * E
        for e in range(E):
            hit = jnp.minimum(jnp.where(v[e] == m, one, zero), 1.0 - seen)
            pick1[e] = hit
            seen = seen + hit
        v2 = [jnp.where(pick1[e] > 0.5, neg, v[e]) for e in range(E)]
        m2 = functools.reduce(jnp.maximum, v2)
        seen2 = zero
        for e in range(E):
            hit2 = jnp.minimum(jnp.where(v2[e] == m2, one, zero),
                               1.0 - seen2)
            seen2 = seen2 + hit2
            cf_v[pl.ds(e * CHUNK + j * L, L)] = \
                (pick1[e] + hit2) * (ex[e] * inv)
    for e in range(E):
        pltpu.sync_copy(cf_v.at[pl.ds(e * CHUNK, CHUNK)],
                        coef_hbm.at[pl.ds(e * T + base, CHUNK)])


def _routed_body(x_ref, coef_ref, shg_ref, shu_ref, shdt_ref, wg_ref,
                 wu_ref, wd_ref, out_ref, acc_ref):
    h = pl.program_id(0)
    i = pl.program_id(1)
    x = x_ref[...]
    gs = jax.lax.dot_general(x, shg_ref[...], _NT,
                             preferred_element_type=jnp.float32)
    us = jax.lax.dot_general(x, shu_ref[...], _NT,
                             preferred_element_type=jnp.float32)
    ts = _silu(gs) * us
    shp = jax.lax.dot_general(ts, shdt_ref[...], _NN,
                              preferred_element_type=jnp.float32)
    r2 = lax.broadcasted_iota(jnp.int32, (TM, TM), 0)
    c2 = lax.broadcasted_iota(jnp.int32, (TM, TM), 1)
    eye = jnp.where(r2 == c2, 1.0, 0.0).astype(jnp.float32)
    contrib = shp
    for e in range(EH):
        g = jax.lax.dot_general(x, wg_ref[e], _NT,
                                preferred_element_type=jnp.float32)
        u = jax.lax.dot_general(x, wu_ref[e], _NT,
                                preferred_element_type=jnp.float32)
        t = _silu(g) * u
        y = jax.lax.dot_general(t, wd_ref[e], _NT,
                                preferred_element_type=jnp.float32)
        ccol = jax.lax.dot_general(eye, coef_ref[e], _NT,
                                   precision=jax.lax.Precision.HIGHEST,
                                   preferred_element_type=jnp.float32)
        contrib = contrib + ccol * y
    rows = pl.ds(i * TM, TM)
    tot = jnp.where(h == 0, contrib, acc_ref[rows, :] + contrib)
    acc_ref[rows, :] = tot
    out_ref[...] = tot


DSQ = 1408 // NH


def _routed_call(x, coefs, shg, shu, shdt, wg, wu, wd):
    return pl.pallas_call(
        _routed_body,
        grid=(NH, T // TM),
        in_specs=[
            pl.BlockSpec((TM, D_MODEL), lambda h, i: (i, 0)),
            pl.BlockSpec((EH, 1, TM), lambda h, i: (h, 0, i)),
            pl.BlockSpec((DSQ, D_MODEL), lambda h, i: (h, 0)),
            pl.BlockSpec((DSQ, D_MODEL), lambda h, i: (h, 0)),
            pl.BlockSpec((DSQ, D_MODEL), lambda h, i: (h, 0)),
            pl.BlockSpec((EH, D_FF, D_MODEL), lambda h, i: (h, 0, 0)),
            pl.BlockSpec((EH, D_FF, D_MODEL), lambda h, i: (h, 0, 0)),
            pl.BlockSpec((EH, D_MODEL, D_FF), lambda h, i: (h, 0, 0)),
        ],
        out_specs=pl.BlockSpec((TM, D_MODEL), lambda h, i: (i, 0)),
        out_shape=jax.ShapeDtypeStruct((T, D_MODEL), jnp.float32),
        scratch_shapes=[pltpu.VMEM((T, D_MODEL), jnp.float32)],
    )(x, coefs, shg, shu, shdt, wg, wu, wd)


def kernel(hidden_states, gate_w, w_gate, w_up, w_down, sh_gate, sh_up,
           sh_down):
    bsz, seq_len, h = hidden_states.shape
    x = hidden_states.reshape(-1, h)
    d_sh = sh_gate.shape[0]

    logt = pl.pallas_call(
        _gate_body,
        grid=(T // TA,),
        in_specs=[
            pl.BlockSpec((TA, D_MODEL), lambda i: (i, 0)),
            pl.BlockSpec((E, D_MODEL), lambda i: (0, 0)),
        ],
        out_specs=pl.BlockSpec((E, TA), lambda i: (0, i)),
        out_shape=jax.ShapeDtypeStruct((E, T), jnp.float32),
    )(x, gate_w)
    shdt = jnp.transpose(sh_down)

    router = functools.partial(
        pl.kernel,
        mesh=plsc.VectorSubcoreMesh(core_axis_name="c", subcore_axis_name="s",
                                    num_cores=2, num_subcores=16),
        out_type=jax.ShapeDtypeStruct((T * E,), jnp.float32),
        scratch_types=[
            pltpu.VMEM((E * CHUNK,), jnp.float32),
            pltpu.VMEM((CHUNK * E,), jnp.float32),
        ],
    )(_router_body)
    coefs_t = router(logt.reshape(E * T)).reshape(E, 1, T)

    out = _routed_call(x, coefs_t, sh_gate, sh_up, shdt,
                       w_gate, w_up, w_down)
    return out.reshape(bsz, seq_len, h).astype(hidden_states.dtype)


# speedup vs baseline: 1.1264x; 1.0061x over previous
import functools

import jax
import jax.numpy as jnp
from jax import lax
from jax.experimental import pallas as pl
from jax.experimental.pallas import tpu as pltpu
from jax.experimental.pallas import tpu_sc as plsc

E = 8
EH = 2
NH = E // EH
D_MODEL = 1024
D_FF = 704
T = 2048
TA = 512
TM = 256
NW = 32
CHUNK = T // NW
L = 16

_NT = (((1,), (1,)), ((), ()))
_NN = (((1,), (0,)), ((), ()))


def _silu(x):
    return x * jax.nn.sigmoid(x)


def _gate_body(x_ref, gate_ref, logt_ref):
    logt_ref[...] = jax.lax.dot_general(
        gate_ref[...], x_ref[...], _NT, preferred_element_type=jnp.float32)


def _router_body(logt_hbm, coef_hbm, lg_v, cf_v):
    wid = lax.axis_index("s") * 2 + lax.axis_index("c")
    base = wid * CHUNK
    for e in range(E):
        pltpu.sync_copy(logt_hbm.at[pl.ds(e * T + base, CHUNK)],
                        lg_v.at[pl.ds(e * CHUNK, CHUNK)])
    one = jnp.full((L,), 1.0, dtype=jnp.float32)
    zero = jnp.full((L,), 0.0, dtype=jnp.float32)
    neg = jnp.full((L,), -3e38, dtype=jnp.float32)
    for j in range(CHUNK // L):
        v = [lg_v[pl.ds(e * CHUNK + j * L, L)] for e in range(E)]
        m = functools.reduce(jnp.maximum, v)
        ex = [jnp.exp(ve - m) for ve in v]
        s = functools.reduce(lambda a, b: a + b, ex)
        inv = 1.0 / s
        seen = zero
        pick1 = [None] * E
        for e in range(E):
            hit = jnp.minimum(jnp.where(v[e] == m, one, zero), 1.0 - seen)
            pick1[e] = hit
            seen = seen + hit
        v2 = [jnp.where(pick1[e] > 0.5, neg, v[e]) for e in range(E)]
        m2 = functools.reduce(jnp.maximum, v2)
        seen2 = zero
        for e in range(E):
            hit2 = jnp.minimum(jnp.where(v2[e] == m2, one, zero),
                               1.0 - seen2)
            seen2 = seen2 + hit2
            cf_v[pl.ds(e * CHUNK + j * L, L)] = \
                (pick1[e] + hit2) * (ex[e] * inv)
    for e in range(E):
        pltpu.sync_copy(cf_v.at[pl.ds(e * CHUNK, CHUNK)],
                        coef_hbm.at[pl.ds(e * T + base, CHUNK)])


def _routed_body(x_ref, coef_ref, shg_ref, shu_ref, shdt_ref, wg_ref,
                 wu_ref, wd_ref, out_ref, acc_ref):
    h = pl.program_id(0)
    i = pl.program_id(1)
    x = x_ref[...]
    gs = jax.lax.dot_general(x, shg_ref[...], _NT,
                             preferred_element_type=jnp.float32)
    us = jax.lax.dot_general(x, shu_ref[...], _NT,
                             preferred_element_type=jnp.float32)
    ts = _silu(gs) * us
    shp = jax.lax.dot_general(ts, shdt_ref[...], _NN,
                              preferred_element_type=jnp.float32)
    r2 = lax.broadcasted_iota(jnp.int32, (TM, TM), 0)
    c2 = lax.broadcasted_iota(jnp.int32, (TM, TM), 1)
    eye = jnp.where(r2 == c2, 1.0, 0.0).astype(jnp.float32)
    contrib = shp
    for e in range(EH):
        g = jax.lax.dot_general(x, wg_ref[e], _NT,
                                preferred_element_type=jnp.float32)
        u = jax.lax.dot_general(x, wu_ref[e], _NT,
                                preferred_element_type=jnp.float32)
        t = _silu(g) * u
        y = jax.lax.dot_general(t, wd_ref[e], _NT,
                                preferred_element_type=jnp.float32)
        ccol = jax.lax.dot_general(eye, coef_ref[e], _NT,
                                   precision=jax.lax.Precision.HIGHEST,
                                   preferred_element_type=jnp.float32)
        contrib = contrib + ccol * y
    rows = pl.ds(i * TM, TM)
    tot = jnp.where(h == 0, contrib, acc_ref[rows, :] + contrib)
    acc_ref[rows, :] = tot
    out_ref[...] = tot


DSQ = 1408 // NH


def _routed_call(x, coefs, shg, shu, shdt, wg, wu, wd):
    return pl.pallas_call(
        _routed_body,
        grid=(NH, T // TM),
        in_specs=[
            pl.BlockSpec((TM, D_MODEL), lambda h, i: (i, 0)),
            pl.BlockSpec((EH, 1, TM), lambda h, i: (h, 0, i)),
            pl.BlockSpec((DSQ, D_MODEL), lambda h, i: (h, 0)),
            pl.BlockSpec((DSQ, D_MODEL), lambda h, i: (h, 0)),
            pl.BlockSpec((DSQ, D_MODEL), lambda h, i: (h, 0)),
            pl.BlockSpec((EH, D_FF, D_MODEL), lambda h, i: (h, 0, 0)),
            pl.BlockSpec((EH, D_FF, D_MODEL), lambda h, i: (h, 0, 0)),
            pl.BlockSpec((EH, D_MODEL, D_FF), lambda h, i: (h, 0, 0)),
        ],
        out_specs=pl.BlockSpec((TM, D_MODEL), lambda h, i: (i, 0)),
        out_shape=jax.ShapeDtypeStruct((T, D_MODEL), jnp.float32),
        scratch_shapes=[pltpu.VMEM((T, D_MODEL), jnp.float32)],
    )(x, coefs, shg, shu, shdt, wg, wu, wd)


def kernel(hidden_states, gate_w, w_gate, w_up, w_down, sh_gate, sh_up,
           sh_down):
    bsz, seq_len, h = hidden_states.shape
    x = hidden_states.reshape(-1, h)
    d_sh = sh_gate.shape[0]

    logt = pl.pallas_call(
        _gate_body,
        grid=(T // TA,),
        in_specs=[
            pl.BlockSpec((TA, D_MODEL), lambda i: (i, 0)),
            pl.BlockSpec((E, D_MODEL), lambda i: (0, 0)),
        ],
        out_specs=pl.BlockSpec((E, TA), lambda i: (0, i)),
        out_shape=jax.ShapeDtypeStruct((E, T), jnp.float32),
    )(x, gate_w)
    shdt = jnp.transpose(sh_down)

    router = functools.partial(
        pl.kernel,
        mesh=plsc.VectorSubcoreMesh(core_axis_name="c", subcore_axis_name="s",
                                    num_cores=2, num_subcores=16),
        out_type=jax.ShapeDtypeStruct((T * E,), jnp.float32),
        scratch_types=[
            pltpu.VMEM((E * CHUNK,), jnp.float32),
            pltpu.VMEM((CHUNK * E,), jnp.float32),
        ],
    )(_router_body)
    coefs_t = router(logt.reshape(E * T)).reshape(E, 1, T)

    out = _routed_call(x, coefs_t, sh_gate, sh_up, shdt,
                       w_gate, w_up, w_down)
    return out.reshape(bsz, seq_len, h).astype(hidden_states.dtype)
